# baseline (speedup 1.0000x reference)
import functools

import jax
import jax.numpy as jnp
from jax.experimental import pallas as pl
from jax.experimental.pallas import tpu as pltpu
from jax.experimental.pallas import tpu_sc as plsc

N = 4096
F = 256
L = 5532
Q = 5000
SCALAR = 30.0
BC = 512
NBP = (L + Q + BC - 1) // BC
PAD = NBP * BC - (L + Q)
NEG = -1e30

_NC = 2
_NS = 16
_NW = _NC * _NS
_BPW = N // _NW


def _gather_body(table_hbm, idx_hbm, out_hbm, idx_v, rows_v, sem):
    wid = jax.lax.axis_index("s") * _NC + jax.lax.axis_index("c")
    base = wid * _BPW
    pltpu.sync_copy(idx_hbm.at[pl.ds(base, _BPW)], idx_v)
    pltpu.async_copy(table_hbm.at[idx_v], rows_v, sem).wait()
    pltpu.sync_copy(rows_v, out_hbm.at[pl.ds(base, _BPW)])


def _sc_gather(table, idx):
    mesh = plsc.VectorSubcoreMesh(core_axis_name="c", subcore_axis_name="s")
    return pl.kernel(
        _gather_body,
        mesh=mesh,
        out_type=jax.ShapeDtypeStruct((N, F), jnp.float32),
        scratch_types=[
            pltpu.VMEM((_BPW,), jnp.int32),
            pltpu.VMEM((_BPW, F), jnp.float32),
            pltpu.SemaphoreType.DMA,
        ],
    )(table, idx)


def _lse_body(x_ref, t_ref, m_out, s_out, m_s, s_s):
    j = pl.program_id(0)

    @pl.when(j == 0)
    def _init():
        m_s[...] = jnp.full((N, 1), NEG, dtype=jnp.float32)
        s_s[...] = jnp.zeros((N, 1), dtype=jnp.float32)

    logits = jax.lax.dot_general(
        x_ref[...], t_ref[...], (((1,), (1,)), ((), ())),
        preferred_element_type=jnp.float32)

    m_old = m_s[...]
    m_new = jnp.maximum(m_old, jnp.max(logits, axis=1, keepdims=True))
    s_s[...] = (s_s[...] * jnp.exp(SCALAR * (m_old - m_new))
                + jnp.sum(jnp.exp(SCALAR * (logits - m_new)),
                          axis=1, keepdims=True))
    m_s[...] = m_new

    @pl.when(j == NBP - 1)
    def _finish():
        m = m_s[...]
        m_out[...] = m
        s_out[...] = s_s[...] - PAD * jnp.exp(-SCALAR * m)


@functools.partial(jax.jit, donate_argnums=())
def _oim_loss(inputs, label_f, safe_idx, lut, cq):
    tables = jnp.concatenate(
        [lut, cq, jnp.zeros((PAD, F), jnp.float32)], axis=0)

    m, s = pl.pallas_call(
        _lse_body,
        grid=(NBP,),
        in_specs=[
            pl.BlockSpec((N, F), lambda j: (0, 0)),
            pl.BlockSpec((BC, F), lambda j: (j, 0)),
        ],
        out_specs=[
            pl.BlockSpec((N, 1), lambda j: (0, 0)),
            pl.BlockSpec((N, 1), lambda j: (0, 0)),
        ],
        out_shape=[
            jax.ShapeDtypeStruct((N, 1), jnp.float32),
            jax.ShapeDtypeStruct((N, 1), jnp.float32),
        ],
        scratch_shapes=[
            pltpu.VMEM((N, 1), jnp.float32),
            pltpu.VMEM((N, 1), jnp.float32),
        ],
        compiler_params=pltpu.CompilerParams(
            dimension_semantics=("arbitrary",)),
    )(inputs, tables)

    gathered = _sc_gather(lut, safe_idx)

    out = pl.pallas_call(
        _combine_body,
        in_specs=[
            pl.BlockSpec((N, F), lambda: (0, 0)),
            pl.BlockSpec((N, F), lambda: (0, 0)),
            pl.BlockSpec((N, 1), lambda: (0, 0)),
            pl.BlockSpec((N, 1), lambda: (0, 0)),
            pl.BlockSpec((N, 1), lambda: (0, 0)),
        ],
        out_specs=pl.BlockSpec((1, 1), lambda: (0, 0)),
        out_shape=jax.ShapeDtypeStruct((1, 1), jnp.float32),
    )(inputs, gathered, m, s, label_f)
    return out[0, 0]


def _combine_body(x_ref, g_ref, m_ref, s_ref, lbl_ref, out_ref):
    d = jnp.sum(x_ref[...] * g_ref[...], axis=1, keepdims=True)
    valid = lbl_ref[...] >= 0.0
    nll = SCALAR * (m_ref[...] - d) + jnp.log(s_ref[...])
    loss_sum = jnp.sum(jnp.where(valid, nll, 0.0), keepdims=True)
    cnt = jnp.sum(valid.astype(jnp.float32), keepdims=True)
    out_ref[...] = loss_sum / jnp.maximum(cnt, 1.0)


def kernel(inputs, roi_label, detectionscore, lut, cq):
    label = roi_label.reshape(-1) - 1
    label_f = label.astype(jnp.float32).reshape(-1, 1)
    safe_idx = jnp.maximum(label, 0).astype(jnp.int32)
    loss = _oim_loss(inputs, label_f, safe_idx, lut, cq)
    return (loss, lut)


# speedup vs baseline: 1.7439x; 1.7439x over previous
import jax
import jax.numpy as jnp
from jax.experimental import pallas as pl
from jax.experimental.pallas import tpu as pltpu

N = 4096
F = 256
L = 5532
Q = 5000
SCALAR = 30.0
BC = 512
NBP = (L + Q + BC - 1) // BC
PAD = NBP * BC - (L + Q)
NEG = -1e30


def _oim_body(x_ref, t_ref, lbl_ref, out_ref, m_s, s_s, g_s):
    j = pl.program_id(0)

    @pl.when(j == 0)
    def _init():
        m_s[...] = jnp.full((N, 1), NEG, dtype=jnp.float32)
        s_s[...] = jnp.zeros((N, 1), dtype=jnp.float32)
        g_s[...] = jnp.zeros((N, 1), dtype=jnp.float32)

    logits = jax.lax.dot_general(
        x_ref[...], t_ref[...], (((1,), (1,)), ((), ())),
        preferred_element_type=jnp.float32)

    col = j * BC + jax.lax.broadcasted_iota(jnp.int32, (1, BC), 1)
    hit = col == lbl_ref[...].astype(jnp.int32)
    g_s[...] += jnp.sum(jnp.where(hit, logits, 0.0), axis=1, keepdims=True)

    m_old = m_s[...]
    m_new = jnp.maximum(m_old, jnp.max(logits, axis=1, keepdims=True))
    s_s[...] = (s_s[...] * jnp.exp(SCALAR * (m_old - m_new))
                + jnp.sum(jnp.exp(SCALAR * (logits - m_new)),
                          axis=1, keepdims=True))
    m_s[...] = m_new

    @pl.when(j == NBP - 1)
    def _finish():
        m = m_s[...]
        s = s_s[...] - PAD * jnp.exp(-SCALAR * m)
        valid = lbl_ref[...] >= 0.0
        nll = SCALAR * (m - g_s[...]) + jnp.log(s)
        loss_sum = jnp.sum(jnp.where(valid, nll, 0.0), keepdims=True)
        cnt = jnp.sum(valid.astype(jnp.float32), keepdims=True)
        out_ref[...] = loss_sum / jnp.maximum(cnt, 1.0)


@jax.jit
def _oim_loss(inputs, label_f, lut, cq):
    tables = jnp.concatenate(
        [lut, cq, jnp.zeros((PAD, F), jnp.float32)], axis=0)

    out = pl.pallas_call(
        _oim_body,
        grid=(NBP,),
        in_specs=[
            pl.BlockSpec((N, F), lambda j: (0, 0)),
            pl.BlockSpec((BC, F), lambda j: (j, 0)),
            pl.BlockSpec((N, 1), lambda j: (0, 0)),
        ],
        out_specs=pl.BlockSpec((1, 1), lambda j: (0, 0)),
        out_shape=jax.ShapeDtypeStruct((1, 1), jnp.float32),
        scratch_shapes=[
            pltpu.VMEM((N, 1), jnp.float32),
            pltpu.VMEM((N, 1), jnp.float32),
            pltpu.VMEM((N, 1), jnp.float32),
        ],
        compiler_params=pltpu.CompilerParams(
            dimension_semantics=("arbitrary",)),
    )(inputs, tables, label_f)
    return out[0, 0]


def kernel(inputs, roi_label, detectionscore, lut, cq):
    label_f = (roi_label.reshape(-1, 1) - 1).astype(jnp.float32)
    loss = _oim_loss(inputs, label_f, lut, cq)
    return (loss, lut)


# speedup vs baseline: 2.2869x; 1.3114x over previous
import jax
import jax.numpy as jnp
from jax.experimental import pallas as pl
from jax.experimental.pallas import tpu as pltpu

N = 4096
F = 256
L = 5532
Q = 5000
SCALAR = 30.0
BC = 512
NLB = (L + BC - 1) // BC
NQB = (Q + BC - 1) // BC
NB = NLB + NQB
NEG = -1e30


def _oim_body(x_ref, lut_ref, cq_ref, lbl_ref, out_ref, t_s, m_s, s_s, g_s):
    j = pl.program_id(0)

    @pl.when(j == 0)
    def _init():
        m_s[...] = jnp.full((1, N), NEG, dtype=jnp.float32)
        s_s[...] = jnp.zeros((1, N), dtype=jnp.float32)
        g_s[...] = jnp.zeros((1, N), dtype=jnp.float32)

    is_lut = j < NLB

    @pl.when(is_lut)
    def _pick_lut():
        t_s[...] = lut_ref[...]

    @pl.when(jnp.logical_not(is_lut))
    def _pick_cq():
        t_s[...] = cq_ref[...]

    logits = jax.lax.dot_general(
        t_s[...], x_ref[...], (((1,), (1,)), ((), ())),
        preferred_element_type=jnp.float32)

    base = jnp.where(is_lut, j * BC, L + (j - NLB) * BC)
    limit = jnp.where(is_lut, L, L + Q)
    col = base + jax.lax.broadcasted_iota(jnp.int32, (BC, 1), 0)
    masked = jnp.where(col < limit, logits, NEG)

    hit = col == lbl_ref[...].astype(jnp.int32)
    g_s[...] += jnp.sum(jnp.where(hit, masked, 0.0), axis=0, keepdims=True)

    m_old = m_s[...]
    m_new = jnp.maximum(m_old, jnp.max(masked, axis=0, keepdims=True))
    s_s[...] = (s_s[...] * jnp.exp(SCALAR * (m_old - m_new))
                + jnp.sum(jnp.exp(SCALAR * (masked - m_new)),
                          axis=0, keepdims=True))
    m_s[...] = m_new

    @pl.when(j == NB - 1)
    def _finish():
        valid = lbl_ref[...] >= 0.0
        nll = SCALAR * (m_s[...] - g_s[...]) + jnp.log(s_s[...])
        loss_sum = jnp.sum(jnp.where(valid, nll, 0.0), keepdims=True)
        cnt = jnp.sum(valid.astype(jnp.float32), keepdims=True)
        out_ref[...] = loss_sum / jnp.maximum(cnt, 1.0)


@jax.jit
def _oim_loss(inputs, label_f, lut, cq):
    out = pl.pallas_call(
        _oim_body,
        grid=(NB,),
        in_specs=[
            pl.BlockSpec((N, F), lambda j: (0, 0)),
            pl.BlockSpec((BC, F), lambda j: (jnp.minimum(j, NLB - 1), 0)),
            pl.BlockSpec((BC, F), lambda j: (jnp.maximum(j - NLB, 0), 0)),
            pl.BlockSpec((1, N), lambda j: (0, 0)),
        ],
        out_specs=pl.BlockSpec((1, 1), lambda j: (0, 0)),
        out_shape=jax.ShapeDtypeStruct((1, 1), jnp.float32),
        scratch_shapes=[
            pltpu.VMEM((BC, F), jnp.float32),
            pltpu.VMEM((1, N), jnp.float32),
            pltpu.VMEM((1, N), jnp.float32),
            pltpu.VMEM((1, N), jnp.float32),
        ],
        compiler_params=pltpu.CompilerParams(
            dimension_semantics=("arbitrary",)),
    )(inputs, lut, cq, label_f)
    return out[0, 0]


def kernel(inputs, roi_label, detectionscore, lut, cq):
    label_f = (roi_label.reshape(1, -1) - 1).astype(jnp.float32)
    loss = _oim_loss(inputs, label_f, lut, cq)
    return (loss, lut)
